# baseline (speedup 1.0000x reference)
import numpy as np

import jax
import jax.numpy as jnp
from jax.experimental import pallas as pl
from jax.experimental.pallas import tpu as pltpu

_TB = 128


def _fused_kernel(x_ref,
                  m1_ref,
                  b1_ref,
                  m2_ref,
                  b2_ref,
                  fw1_ref, fb1_ref,
                  fw2_ref, fb2_ref,
                  fw3_ref, fb3_ref,
                  o_ref):
    f32 = jnp.float32
    vt = jnp.transpose(x_ref[...])
    vt3 = vt.reshape(32, 32, _TB)
    x = jnp.concatenate([vt3[:, k, :] for k in range(32)], axis=1)

    a1s = jnp.concatenate(
        [x[:, 0:3840], x[:, 128:3968], x[:, 256:4096]], axis=0)
    c1 = jnp.dot(m1_ref[...], a1s, preferred_element_type=f32)

    px = jnp.concatenate(
        [jnp.maximum(c1[:, 256 * k:256 * k + 128],
                     c1[:, 256 * k + 128:256 * k + 256])
         for k in range(15)], axis=1)
    py = jnp.concatenate(
        [jnp.maximum(px[12 * t:12 * t + 6, :], px[12 * t + 6:12 * t + 12, :])
         for t in range(15)], axis=0)
    a2 = jnp.maximum(py + b1_ref[...], 0.0)

    c2 = (jnp.dot(m2_ref[0, :, :], a2[:, 0:1664],
                  preferred_element_type=f32)
          + jnp.dot(m2_ref[1, :, :], a2[:, 128:1792],
                    preferred_element_type=f32)
          + jnp.dot(m2_ref[2, :, :], a2[:, 256:1920],
                    preferred_element_type=f32))

    p2y = jnp.concatenate(
        [jnp.maximum(c2[32 * t:32 * t + 16, :], c2[32 * t + 16:32 * t + 32, :])
         for t in range(6)], axis=0)
    hf = jnp.concatenate(
        [jnp.maximum(p2y[:, 256 * k:256 * k + 128],
                     p2y[:, 256 * k + 128:256 * k + 256])
         for k in range(6)], axis=0)
    hf = jnp.maximum(hf + b2_ref[...], 0.0)

    h3 = jnp.maximum(jnp.dot(fw1_ref[...], hf,
                             preferred_element_type=f32) + fb1_ref[...], 0.0)
    h4 = jnp.maximum(jnp.dot(fw2_ref[...], h3,
                             preferred_element_type=f32) + fb2_ref[...], 0.0)
    z = jnp.dot(fw3_ref[...], h4,
                preferred_element_type=f32) + fb3_ref[...]
    o_ref[...] = 1.0 / (1.0 + jnp.exp(-z))


def _build_conv1_matrix(w1):
    co, y, i, j = np.meshgrid(np.arange(6), np.arange(30), np.arange(3),
                              np.arange(3), indexing="ij")
    rows = (y * 6 + co).ravel()
    cols = (j * 32 + y + i).ravel()
    vals = w1[co.ravel(), 0, i.ravel(), j.ravel()].astype(jnp.float32)
    return jnp.zeros((180, 96), jnp.float32).at[rows, cols].set(vals)


def _build_conv2_matrix(w2):
    co, y, ci, i, j = np.meshgrid(np.arange(16), np.arange(13), np.arange(6),
                                  np.arange(3), np.arange(3), indexing="ij")
    rows = (y * 16 + co).ravel()
    cols = ((y + i) * 6 + ci).ravel()
    vals = w2[co.ravel(), ci.ravel(), i.ravel(), j.ravel()].astype(jnp.float32)
    return jnp.zeros((3, 208, 90), jnp.float32).at[j.ravel(), rows, cols].set(vals)


def kernel(conv1_w, conv1_b, conv2_w, conv2_b, fc1_w, fc1_b,
           fc2_w, fc2_b, fc3_w, fc3_b, x_nchw):
    f32 = jnp.float32
    n = x_nchw.shape[0]
    n_pad = ((n + _TB - 1) // _TB) * _TB
    t = n_pad // _TB

    x = jnp.asarray(x_nchw, f32).reshape(n, 32 * 32)
    xa = jnp.pad(x, ((0, n_pad - n), (0, 0)))

    m1 = _build_conv1_matrix(conv1_w)
    b1c = jnp.tile(conv1_b.astype(f32), (15,)).reshape(90, 1)
    m2 = _build_conv2_matrix(conv2_w)
    b2c = jnp.tile(conv2_b.astype(f32), (36,)).reshape(576, 1)
    fw1 = fc1_w.reshape(128, 16, 6, 6).transpose(0, 3, 2, 1).reshape(128, 576)
    fw1 = fw1.astype(f32)
    fb1 = fc1_b.reshape(128, 1).astype(f32)
    fw2 = fc2_w.astype(f32)
    fb2 = fc2_b.reshape(64, 1).astype(f32)
    fw3 = fc3_w.astype(f32)
    fb3 = fc3_b.reshape(1, 1).astype(f32)

    out = pl.pallas_call(
        _fused_kernel,
        out_shape=jax.ShapeDtypeStruct((1, n_pad), f32),
        grid=(t,),
        in_specs=[
            pl.BlockSpec((_TB, 1024), lambda i: (i, 0)),
            pl.BlockSpec((180, 96), lambda i: (0, 0)),
            pl.BlockSpec((90, 1), lambda i: (0, 0)),
            pl.BlockSpec((3, 208, 90), lambda i: (0, 0, 0)),
            pl.BlockSpec((576, 1), lambda i: (0, 0)),
            pl.BlockSpec((128, 576), lambda i: (0, 0)),
            pl.BlockSpec((128, 1), lambda i: (0, 0)),
            pl.BlockSpec((64, 128), lambda i: (0, 0)),
            pl.BlockSpec((64, 1), lambda i: (0, 0)),
            pl.BlockSpec((1, 64), lambda i: (0, 0)),
            pl.BlockSpec((1, 1), lambda i: (0, 0)),
        ],
        out_specs=pl.BlockSpec((1, _TB), lambda i: (0, i)),
        compiler_params=pltpu.CompilerParams(
            dimension_semantics=("parallel",)),
    )(xa, m1, b1c, m2, b2c, fw1, fb1, fw2, fb2, fw3, fb3)

    return jnp.transpose(out[:, :n])


# speedup vs baseline: 2.1816x; 2.1816x over previous
import numpy as np

import jax
import jax.numpy as jnp
from jax.experimental import pallas as pl
from jax.experimental.pallas import tpu as pltpu

_TB = 128


def _fused_kernel(x_ref,
                  m1_ref,
                  b1_ref,
                  m2_ref,
                  b2_ref,
                  fw1_ref, fb1_ref,
                  fw2_ref, fb2_ref,
                  fw3_ref, fb3_ref,
                  o_ref):
    f32 = jnp.float32
    vt = jnp.transpose(x_ref[...])
    vt3 = vt.reshape(32, 32, _TB)
    x = jnp.concatenate([vt3[:, k, :] for k in range(32)], axis=1)

    a1s = jnp.concatenate(
        [x[:, 0:3840], x[:, 128:3968], x[:, 256:4096]], axis=0)
    c1 = jnp.dot(m1_ref[...], a1s, preferred_element_type=f32)

    px = jnp.concatenate(
        [jnp.maximum(c1[:, 256 * k:256 * k + 128],
                     c1[:, 256 * k + 128:256 * k + 256])
         for k in range(15)], axis=1)
    py = jnp.concatenate(
        [jnp.maximum(px[12 * t:12 * t + 6, :], px[12 * t + 6:12 * t + 12, :])
         for t in range(15)], axis=0)
    a2 = jnp.maximum(py + b1_ref[...], 0.0)

    c2 = (jnp.dot(m2_ref[0, :, :], a2[:, 0:1664],
                  preferred_element_type=f32)
          + jnp.dot(m2_ref[1, :, :], a2[:, 128:1792],
                    preferred_element_type=f32)
          + jnp.dot(m2_ref[2, :, :], a2[:, 256:1920],
                    preferred_element_type=f32))

    p2y = jnp.concatenate(
        [jnp.maximum(c2[32 * t:32 * t + 16, :], c2[32 * t + 16:32 * t + 32, :])
         for t in range(6)], axis=0)
    hf = jnp.concatenate(
        [jnp.maximum(p2y[:, 256 * k:256 * k + 128],
                     p2y[:, 256 * k + 128:256 * k + 256])
         for k in range(6)], axis=0)
    hf = jnp.maximum(hf + b2_ref[...], 0.0)

    h3 = jnp.maximum(jnp.dot(fw1_ref[...], hf,
                             preferred_element_type=f32) + fb1_ref[...], 0.0)
    h4 = jnp.maximum(jnp.dot(fw2_ref[...], h3,
                             preferred_element_type=f32) + fb2_ref[...], 0.0)
    z = jnp.dot(fw3_ref[...], h4,
                preferred_element_type=f32) + fb3_ref[...]
    o_ref[...] = 1.0 / (1.0 + jnp.exp(-z))


def _band(n_out, n_in, i):
    b = np.zeros((n_out, n_in), np.float32)
    b[np.arange(n_out), np.arange(n_out) + i] = 1.0
    return b


def _build_conv1_matrix(w1):
    w = w1[:, 0].astype(jnp.float32)
    cols = []
    for j in range(3):
        mj = sum(jnp.asarray(_band(30, 32, i))[:, None, :] * w[None, :, i, j, None]
                 for i in range(3))
        cols.append(mj.reshape(180, 32))
    return jnp.concatenate(cols, axis=1)


def _build_conv2_matrix(w2):
    w = w2.astype(jnp.float32)
    mats = []
    for j in range(3):
        mj = sum(jnp.asarray(_band(13, 15, i))[:, None, :, None]
                 * w[None, :, None, :, i, j]
                 for i in range(3))
        mats.append(mj.reshape(208, 90))
    return jnp.stack(mats, axis=0)


def kernel(conv1_w, conv1_b, conv2_w, conv2_b, fc1_w, fc1_b,
           fc2_w, fc2_b, fc3_w, fc3_b, x_nchw):
    f32 = jnp.float32
    n = x_nchw.shape[0]
    n_pad = ((n + _TB - 1) // _TB) * _TB
    t = n_pad // _TB

    x = jnp.asarray(x_nchw, f32).reshape(n, 32 * 32)
    xa = jnp.pad(x, ((0, n_pad - n), (0, 0)))

    m1 = _build_conv1_matrix(conv1_w)
    b1c = jnp.tile(conv1_b.astype(f32), (15,)).reshape(90, 1)
    m2 = _build_conv2_matrix(conv2_w)
    b2c = jnp.tile(conv2_b.astype(f32), (36,)).reshape(576, 1)
    fw1 = fc1_w.reshape(128, 16, 6, 6).transpose(0, 3, 2, 1).reshape(128, 576)
    fw1 = fw1.astype(f32)
    fb1 = fc1_b.reshape(128, 1).astype(f32)
    fw2 = fc2_w.astype(f32)
    fb2 = fc2_b.reshape(64, 1).astype(f32)
    fw3 = fc3_w.astype(f32)
    fb3 = fc3_b.reshape(1, 1).astype(f32)

    out = pl.pallas_call(
        _fused_kernel,
        out_shape=jax.ShapeDtypeStruct((1, n_pad), f32),
        grid=(t,),
        in_specs=[
            pl.BlockSpec((_TB, 1024), lambda i: (i, 0)),
            pl.BlockSpec((180, 96), lambda i: (0, 0)),
            pl.BlockSpec((90, 1), lambda i: (0, 0)),
            pl.BlockSpec((3, 208, 90), lambda i: (0, 0, 0)),
            pl.BlockSpec((576, 1), lambda i: (0, 0)),
            pl.BlockSpec((128, 576), lambda i: (0, 0)),
            pl.BlockSpec((128, 1), lambda i: (0, 0)),
            pl.BlockSpec((64, 128), lambda i: (0, 0)),
            pl.BlockSpec((64, 1), lambda i: (0, 0)),
            pl.BlockSpec((1, 64), lambda i: (0, 0)),
            pl.BlockSpec((1, 1), lambda i: (0, 0)),
        ],
        out_specs=pl.BlockSpec((1, _TB), lambda i: (0, i)),
        compiler_params=pltpu.CompilerParams(
            dimension_semantics=("parallel",)),
    )(xa, m1, b1c, m2, b2c, fw1, fb1, fw2, fb2, fw3, fb3)

    return jnp.transpose(out[:, :n])
